# Initial kernel scaffold; baseline (speedup 1.0000x reference)
import jax
import jax.numpy as jnp
from jax.experimental import pallas as pl


def kernel(user_id, gender, status, regis_date, history, voting, favourite, user_table, gender_table, status_table, rgst_table, hsty_table, vote_table, favr_table):
    raise NotImplementedError("write your pallas kernel here")


# speedup vs baseline: 6.7751x; 6.7751x over previous
import functools

import jax
import jax.numpy as jnp
import numpy as np
from jax import lax
from jax.experimental import pallas as pl
from jax.experimental.pallas import tpu as pltpu
from jax.experimental.pallas import tpu_sc as plsc

B = 16384
D = 32
OUT_COLS = 228
NUM_BUCKETS = 1000

_info = plsc.get_sparse_core_info()
NC = _info.num_cores
NS = _info.num_subcores
L = _info.num_lanes
NW = NC * NS
B_PER_W = B // NW
CHUNK = 64
N_CHUNKS = B_PER_W // CHUNK

_DEN = float(np.sqrt(np.float32(1.0 / 12.0 + 1e-7)))

_FIELD_COLS = (0, 32, 64, 96, 129, 162, 195)
_NORM_COLS = (128, 161, 194, 227)


def _idx_compute(uid_v, gv_v, sv_v, uq_v, ucol_v, gsidx_v):
    for j in range(CHUNK // L):
        s16 = pl.ds(j * L, L)
        uid = uid_v[s16]
        uq_v[s16] = jax.lax.shift_right_logical(uid, 2)
        ucol_v[s16] = (uid & 3) * D
        gsidx_v[s16] = gv_v[s16] * 8 + sv_v[s16]


def _bucket_norm_compute(xb_v, bnd_v, bidx_refs, stag_v):
    for f in range(4):
        for j in range(CHUNK // L):
            x = xb_v[f, pl.ds(j * L, L)]
            c = x * np.float32(NUM_BUCKETS - 1)
            t = c.astype(jnp.int32)
            e = t + jnp.where(t.astype(jnp.float32) < c, 1, 0)
            p = jnp.minimum(jnp.maximum(e - 2, 0), NUM_BUCKETS - 4)
            cnt = jnp.where(plsc.load_gather(bnd_v, [p]) < x, 1, 0)
            for k in range(1, 4):
                cnt = cnt + jnp.where(plsc.load_gather(bnd_v, [p + k]) < x, 1, 0)
            bidx_refs[f][pl.ds(j * L, L)] = p + cnt
            nx = (x - np.float32(0.5)) / np.float32(_DEN)
            pos = (jnp.arange(j * L, (j + 1) * L, dtype=jnp.int32) * OUT_COLS
                   + _NORM_COLS[f])
            plsc.store_scatter(stag_v, [pos], nx)


def _body(uid_h, g_h, s_h, rd_h, hy_h, vt_h, fv_h,
          ut_h, gs_h, bk_h, bnd_h,
          out_h,
          bnd_v, uid_v, gv_v, sv_v, uq_v, ucol_v, gsidx_v, xb_v,
          bidx0, bidx1, bidx2, bidx3,
          ud_v, gsd_v, bd0, bd1, bd2, bd3, stag_v, sem):
    wid = lax.axis_index("s") * NC + lax.axis_index("c")
    base = wid * B_PER_W
    pltpu.sync_copy(bnd_h, bnd_v)
    bidx_refs = (bidx0, bidx1, bidx2, bidx3)
    bds = (bd0, bd1, bd2, bd3)

    def _chunk(ci, carry):
        cb = base + ci * CHUNK
        sl = pl.ds(cb, CHUNK)
        pltpu.sync_copy(uid_h.at[sl], uid_v)
        pltpu.sync_copy(g_h.at[sl], gv_v)
        pltpu.sync_copy(s_h.at[sl], sv_v)
        pltpu.sync_copy(rd_h.at[sl], xb_v.at[0])
        pltpu.sync_copy(hy_h.at[sl], xb_v.at[1])
        pltpu.sync_copy(vt_h.at[sl], xb_v.at[2])
        pltpu.sync_copy(fv_h.at[sl], xb_v.at[3])

        _idx_compute(uid_v, gv_v, sv_v, uq_v, ucol_v, gsidx_v)
        cps = [
            pltpu.async_copy(ut_h.at[uq_v], ud_v, sem),
            pltpu.async_copy(gs_h.at[gsidx_v], gsd_v, sem),
        ]
        _bucket_norm_compute(xb_v, bnd_v, bidx_refs, stag_v)
        for f in range(4):
            cps.append(pltpu.async_copy(bk_h.at[bidx_refs[f]], bds[f], sem))
        for cp in cps:
            cp.wait()

        def _repack(r, inner):
            rb = r * OUT_COLS
            cl = ucol_v[pl.ds(r, L)][0]
            stag_v[pl.ds(rb, L)] = ud_v[r, pl.ds(cl, L)]
            stag_v[pl.ds(rb + L, L)] = ud_v[r, pl.ds(cl + L, L)]
            for c in range(0, 2 * D, L):
                stag_v[pl.ds(rb + 32 + c, L)] = gsd_v[r, pl.ds(c, L)]
            for f in range(4):
                col = _FIELD_COLS[3 + f]
                for c in range(0, D, L):
                    stag_v[pl.ds(rb + col + c, L)] = bds[f][r, pl.ds(f * D + c, L)]
            return inner
        lax.fori_loop(0, CHUNK, _repack, 0)

        pltpu.sync_copy(stag_v, out_h.at[pl.ds(cb * OUT_COLS, CHUNK * OUT_COLS)])
        return carry

    lax.fori_loop(0, N_CHUNKS, _chunk, 0)


def kernel(user_id, gender, status, regis_date, history, voting, favourite,
           user_table, gender_table, status_table,
           rgst_table, hsty_table, vote_table, favr_table):
    bnd = jnp.linspace(0.0, 1.0, NUM_BUCKETS).astype(jnp.float32)
    bnd = jnp.concatenate([bnd, jnp.full((8,), 2.0, jnp.float32)])
    ut2 = user_table.reshape(250000, 4 * D)
    gs = jnp.concatenate([jnp.repeat(gender_table, 8, axis=0),
                          jnp.tile(status_table, (3, 1)),
                          jnp.zeros((24, 2 * D), jnp.float32)], axis=1)
    bk = jnp.concatenate([rgst_table, hsty_table, vote_table, favr_table],
                         axis=1)
    mesh = plsc.VectorSubcoreMesh(core_axis_name="c", subcore_axis_name="s")
    run = functools.partial(
        pl.kernel, mesh=mesh,
        compiler_params=pltpu.CompilerParams(needs_layout_passes=False),
        out_type=jax.ShapeDtypeStruct((B * OUT_COLS,), jnp.float32),
        scratch_types=[
            pltpu.VMEM((NUM_BUCKETS + 8,), jnp.float32),
            pltpu.VMEM((CHUNK,), jnp.int32),
            pltpu.VMEM((CHUNK,), jnp.int32),
            pltpu.VMEM((CHUNK,), jnp.int32),
            pltpu.VMEM((CHUNK,), jnp.int32),
            pltpu.VMEM((CHUNK + L,), jnp.int32),
            pltpu.VMEM((CHUNK,), jnp.int32),
            pltpu.VMEM((4, CHUNK), jnp.float32),
            pltpu.VMEM((CHUNK,), jnp.int32),
            pltpu.VMEM((CHUNK,), jnp.int32),
            pltpu.VMEM((CHUNK,), jnp.int32),
            pltpu.VMEM((CHUNK,), jnp.int32),
            pltpu.VMEM((CHUNK, 4 * D), jnp.float32),
            pltpu.VMEM((CHUNK, 4 * D), jnp.float32),
            pltpu.VMEM((CHUNK, 4 * D), jnp.float32),
            pltpu.VMEM((CHUNK, 4 * D), jnp.float32),
            pltpu.VMEM((CHUNK, 4 * D), jnp.float32),
            pltpu.VMEM((CHUNK, 4 * D), jnp.float32),
            pltpu.VMEM((CHUNK * OUT_COLS,), jnp.float32),
            pltpu.SemaphoreType.DMA,
        ],
    )(_body)
    flat = run(user_id.astype(jnp.int32), gender.astype(jnp.int32),
               status.astype(jnp.int32), regis_date, history, voting,
               favourite, ut2, gs, bk, bnd)
    return flat.reshape(B, OUT_COLS)


# speedup vs baseline: 6.7880x; 1.0019x over previous
import functools

import jax
import jax.numpy as jnp
import numpy as np
from jax import lax
from jax.experimental import pallas as pl
from jax.experimental.pallas import tpu as pltpu
from jax.experimental.pallas import tpu_sc as plsc

B = 16384
D = 32
OUT_COLS = 228
NUM_BUCKETS = 1000

_info = plsc.get_sparse_core_info()
NC = _info.num_cores
NS = _info.num_subcores
L = _info.num_lanes
NW = NC * NS
B_PER_W = B // NW
CHUNK = 64
N_CHUNKS = B_PER_W // CHUNK

_DEN = float(np.sqrt(np.float32(1.0 / 12.0 + 1e-7)))

_FIELD_COLS = (0, 32, 64, 96, 129, 162, 195)
_NORM_COLS = (128, 161, 194, 227)


def _idx_compute(uid_v, gv_v, sv_v, uq_v, ucol_v, gsidx_v):
    for j in range(CHUNK // L):
        s16 = pl.ds(j * L, L)
        uid = uid_v[s16]
        uq_v[s16] = jax.lax.shift_right_logical(uid, 2)
        ucol_v[s16] = (uid & 3) * D
        gsidx_v[s16] = gv_v[s16] * 8 + sv_v[s16]


def _bucket_norm_compute(xb_v, bnd_v, bidx_refs, stag_v):
    for f in range(4):
        for j in range(CHUNK // L):
            x = xb_v[f, pl.ds(j * L, L)]
            c = x * np.float32(NUM_BUCKETS - 1)
            t = c.astype(jnp.int32)
            e = t + jnp.where(t.astype(jnp.float32) < c, 1, 0)
            p = jnp.minimum(jnp.maximum(e - 2, 0), NUM_BUCKETS - 4)
            cnt = jnp.where(plsc.load_gather(bnd_v, [p]) < x, 1, 0)
            for k in range(1, 4):
                cnt = cnt + jnp.where(plsc.load_gather(bnd_v, [p + k]) < x, 1, 0)
            bidx_refs[f][pl.ds(j * L, L)] = p + cnt
            nx = (x - np.float32(0.5)) / np.float32(_DEN)
            pos = (jnp.arange(j * L, (j + 1) * L, dtype=jnp.int32) * OUT_COLS
                   + _NORM_COLS[f])
            plsc.store_scatter(stag_v, [pos], nx)


def _body(uid_h, g_h, s_h, rd_h, hy_h, vt_h, fv_h,
          ut_h, gs_h, bk_h, bnd_h,
          out_h,
          bnd_v, uid_v, gv_v, sv_v, uq_v, ucol_v, gsidx_v, xb_v,
          bidx0, bidx1, bidx2, bidx3,
          ud_v, gsd_v, bd0, bd1, bd2, bd3, stag_v, sem):
    wid = lax.axis_index("s") * NC + lax.axis_index("c")
    base = wid * B_PER_W
    pltpu.sync_copy(bnd_h, bnd_v)
    bidx_refs = (bidx0, bidx1, bidx2, bidx3)
    bds = (bd0, bd1, bd2, bd3)

    def _chunk(ci, carry):
        cb = base + ci * CHUNK
        sl = pl.ds(cb, CHUNK)
        pltpu.sync_copy(uid_h.at[sl], uid_v)
        pltpu.sync_copy(g_h.at[sl], gv_v)
        pltpu.sync_copy(s_h.at[sl], sv_v)
        pltpu.sync_copy(rd_h.at[sl], xb_v.at[0])
        pltpu.sync_copy(hy_h.at[sl], xb_v.at[1])
        pltpu.sync_copy(vt_h.at[sl], xb_v.at[2])
        pltpu.sync_copy(fv_h.at[sl], xb_v.at[3])

        _idx_compute(uid_v, gv_v, sv_v, uq_v, ucol_v, gsidx_v)
        cps = [
            pltpu.async_copy(ut_h.at[uq_v], ud_v, sem),
            pltpu.async_copy(gs_h.at[gsidx_v], gsd_v, sem),
        ]
        _bucket_norm_compute(xb_v, bnd_v, bidx_refs, stag_v)
        for f in range(4):
            cps.append(pltpu.async_copy(bk_h.at[bidx_refs[f]], bds[f], sem))
        for cp in cps:
            cp.wait()

        def _repack(r, inner):
            rb = r * OUT_COLS
            cl = ucol_v[pl.ds(r, L)][0]
            stag_v[pl.ds(rb, L)] = ud_v[r, pl.ds(cl, L)]
            stag_v[pl.ds(rb + L, L)] = ud_v[r, pl.ds(cl + L, L)]
            for c in range(0, 2 * D, L):
                stag_v[pl.ds(rb + 32 + c, L)] = gsd_v[r, pl.ds(c, L)]
            for f in range(4):
                col = _FIELD_COLS[3 + f]
                for c in range(0, D, L):
                    stag_v[pl.ds(rb + col + c, L)] = bds[f][r, pl.ds(f * D + c, L)]
            return inner
        lax.fori_loop(0, CHUNK, _repack, 0)

        pltpu.sync_copy(stag_v, out_h.at[pl.ds(cb * OUT_COLS, CHUNK * OUT_COLS)])
        return carry

    lax.fori_loop(0, N_CHUNKS, _chunk, 0)


def kernel(user_id, gender, status, regis_date, history, voting, favourite,
           user_table, gender_table, status_table,
           rgst_table, hsty_table, vote_table, favr_table):
    bnd = jnp.linspace(0.0, 1.0, NUM_BUCKETS).astype(jnp.float32)
    bnd = jnp.concatenate([bnd, jnp.full((8,), 2.0, jnp.float32)])
    ut2 = user_table.reshape(250000, 4 * D) * np.float32(1.0)
    gs = jnp.concatenate([jnp.repeat(gender_table, 8, axis=0),
                          jnp.tile(status_table, (3, 1)),
                          jnp.zeros((24, 2 * D), jnp.float32)], axis=1)
    bk = jnp.concatenate([rgst_table, hsty_table, vote_table, favr_table],
                         axis=1)
    mesh = plsc.VectorSubcoreMesh(core_axis_name="c", subcore_axis_name="s")
    run = functools.partial(
        pl.kernel, mesh=mesh,
        compiler_params=pltpu.CompilerParams(needs_layout_passes=False),
        out_type=jax.ShapeDtypeStruct((B * OUT_COLS,), jnp.float32),
        scratch_types=[
            pltpu.VMEM((NUM_BUCKETS + 8,), jnp.float32),
            pltpu.VMEM((CHUNK,), jnp.int32),
            pltpu.VMEM((CHUNK,), jnp.int32),
            pltpu.VMEM((CHUNK,), jnp.int32),
            pltpu.VMEM((CHUNK,), jnp.int32),
            pltpu.VMEM((CHUNK + L,), jnp.int32),
            pltpu.VMEM((CHUNK,), jnp.int32),
            pltpu.VMEM((4, CHUNK), jnp.float32),
            pltpu.VMEM((CHUNK,), jnp.int32),
            pltpu.VMEM((CHUNK,), jnp.int32),
            pltpu.VMEM((CHUNK,), jnp.int32),
            pltpu.VMEM((CHUNK,), jnp.int32),
            pltpu.VMEM((CHUNK, 4 * D), jnp.float32),
            pltpu.VMEM((CHUNK, 4 * D), jnp.float32),
            pltpu.VMEM((CHUNK, 4 * D), jnp.float32),
            pltpu.VMEM((CHUNK, 4 * D), jnp.float32),
            pltpu.VMEM((CHUNK, 4 * D), jnp.float32),
            pltpu.VMEM((CHUNK, 4 * D), jnp.float32),
            pltpu.VMEM((CHUNK * OUT_COLS,), jnp.float32),
            pltpu.SemaphoreType.DMA,
        ],
    )(_body)
    flat = run(user_id.astype(jnp.int32), gender.astype(jnp.int32),
               status.astype(jnp.int32), regis_date, history, voting,
               favourite, ut2, gs, bk, bnd)
    return flat.reshape(B, OUT_COLS)
